# baseline (speedup 1.0000x reference)
import jax
import jax.numpy as jnp
from jax.experimental import pallas as pl
from jax.experimental.pallas import tpu as pltpu


def _mm_kernel(a_ref, x_ref, o_ref):
    o_ref[...] = jnp.dot(a_ref[...], x_ref[...],
                         preferred_element_type=jnp.float32)


def kernel(x, a):
    n_t, n_s = a.shape
    d = x.shape[1]
    bm = 256
    return pl.pallas_call(
        _mm_kernel,
        grid=(n_t // bm,),
        in_specs=[
            pl.BlockSpec((bm, n_s), lambda i: (i, 0)),
            pl.BlockSpec((n_s, d), lambda i: (0, 0)),
        ],
        out_specs=pl.BlockSpec((bm, d), lambda i: (i, 0)),
        out_shape=jax.ShapeDtypeStruct((n_t, d), jnp.float32),
        compiler_params=pltpu.CompilerParams(
            dimension_semantics=("parallel",),
        ),
    )(a, x)


# speedup vs baseline: 1.0579x; 1.0579x over previous
import jax
import jax.numpy as jnp
from jax.experimental import pallas as pl
from jax.experimental.pallas import tpu as pltpu

_N_CHUNKS = 8


def _mm_kernel(a_hbm, x_ref, o_ref, scr, sems):
    n_rows = scr.shape[0]
    ck = n_rows // _N_CHUNKS
    for i in range(_N_CHUNKS):
        pltpu.make_async_copy(
            a_hbm.at[pl.ds(i * ck, ck), :],
            scr.at[pl.ds(i * ck, ck), :],
            sems.at[i],
        ).start()
    for i in range(_N_CHUNKS):
        pltpu.make_async_copy(
            a_hbm.at[pl.ds(i * ck, ck), :],
            scr.at[pl.ds(i * ck, ck), :],
            sems.at[i],
        ).wait()
        o_ref[pl.ds(i * ck, ck), :] = jnp.dot(
            scr[pl.ds(i * ck, ck), :], x_ref[...],
            preferred_element_type=jnp.float32)


def kernel(x, a):
    n_t, n_s = a.shape
    d = x.shape[1]
    return pl.pallas_call(
        _mm_kernel,
        in_specs=[
            pl.BlockSpec(memory_space=pl.ANY),
            pl.BlockSpec(memory_space=pltpu.MemorySpace.VMEM),
        ],
        out_specs=pl.BlockSpec(memory_space=pltpu.MemorySpace.VMEM),
        out_shape=jax.ShapeDtypeStruct((n_t, d), jnp.float32),
        scratch_shapes=[
            pltpu.VMEM((n_t, n_s), jnp.float32),
            pltpu.SemaphoreType.DMA((_N_CHUNKS,)),
        ],
    )(a, x)
